# baseline (speedup 1.0000x reference)
import jax
import jax.numpy as jnp
from jax.experimental import pallas as pl

_S = 50


def _add_pos_kernel(x_ref, pos_ref, out_ref):
    out_ref[...] = x_ref[...] + pos_ref[: out_ref.shape[1], :][None, :, :]


def kernel(concat_embeddings, pos_table):
    b, s, h = concat_embeddings.shape
    bb = 128
    grid = (b // bb,)
    return pl.pallas_call(
        _add_pos_kernel,
        grid=grid,
        in_specs=[
            pl.BlockSpec((bb, s, h), lambda i: (i, 0, 0)),
            pl.BlockSpec(pos_table.shape, lambda i: (0, 0)),
        ],
        out_specs=pl.BlockSpec((bb, s, h), lambda i: (i, 0, 0)),
        out_shape=jax.ShapeDtypeStruct((b, s, h), concat_embeddings.dtype),
        compiler_params=pltpu_params(),
    )(concat_embeddings, pos_table)


def pltpu_params():
    from jax.experimental.pallas import tpu as pltpu
    return pltpu.CompilerParams(
        dimension_semantics=("arbitrary",),
    )


# speedup vs baseline: 1.0080x; 1.0080x over previous
import jax
import jax.numpy as jnp
from jax.experimental import pallas as pl

_S = 50


def _add_pos_kernel(x_ref, pos_ref, out_ref):
    out_ref[...] = x_ref[...] + pos_ref[: out_ref.shape[1], :][None, :, :]


def kernel(concat_embeddings, pos_table):
    b, s, h = concat_embeddings.shape
    bb = 256
    grid = (b // bb,)
    return pl.pallas_call(
        _add_pos_kernel,
        grid=grid,
        in_specs=[
            pl.BlockSpec((bb, s, h), lambda i: (i, 0, 0)),
            pl.BlockSpec(pos_table.shape, lambda i: (0, 0)),
        ],
        out_specs=pl.BlockSpec((bb, s, h), lambda i: (i, 0, 0)),
        out_shape=jax.ShapeDtypeStruct((b, s, h), concat_embeddings.dtype),
        compiler_params=pltpu_params(),
    )(concat_embeddings, pos_table)


def pltpu_params():
    from jax.experimental.pallas import tpu as pltpu
    return pltpu.CompilerParams(
        dimension_semantics=("parallel",),
    )


# speedup vs baseline: 1.0469x; 1.0386x over previous
import jax
import jax.numpy as jnp
from jax.experimental import pallas as pl
from jax.experimental.pallas import tpu as pltpu

_CB = 128
_NBUF = 4


def _add_pos_kernel(x_hbm, pos_hbm, out_hbm, x_vmem, o_vmem, pos_vmem,
                    in_sems, out_sems, pos_sem):
    nb = x_hbm.shape[0]
    nc = nb // _CB
    s = x_hbm.shape[1]

    pltpu.make_async_copy(pos_hbm, pos_vmem, pos_sem).start()

    def in_copy(i, slot):
        return pltpu.make_async_copy(
            x_hbm.at[pl.ds(i * _CB, _CB)], x_vmem.at[slot], in_sems.at[slot])

    def out_copy(i, slot):
        return pltpu.make_async_copy(
            o_vmem.at[slot], out_hbm.at[pl.ds(i * _CB, _CB)], out_sems.at[slot])

    for k in range(min(_NBUF, nc)):
        in_copy(k, k).start()

    pltpu.make_async_copy(pos_hbm, pos_vmem, pos_sem).wait()
    pos = pos_vmem[:s, :][None, :, :]

    for i in range(nc):
        slot = i % _NBUF
        in_copy(i, slot).wait()
        if i >= _NBUF:
            out_copy(i - _NBUF, slot).wait()
        o_vmem[slot] = x_vmem[slot] + pos
        out_copy(i, slot).start()
        if i + _NBUF < nc:
            in_copy(i + _NBUF, slot).start()

    for i in range(max(nc - _NBUF, 0), nc):
        out_copy(i, i % _NBUF).wait()


def kernel(concat_embeddings, pos_table):
    b, s, h = concat_embeddings.shape
    np_, _ = pos_table.shape
    return pl.pallas_call(
        _add_pos_kernel,
        in_specs=[
            pl.BlockSpec(memory_space=pltpu.MemorySpace.HBM),
            pl.BlockSpec(memory_space=pltpu.MemorySpace.HBM),
        ],
        out_specs=pl.BlockSpec(memory_space=pltpu.MemorySpace.HBM),
        out_shape=jax.ShapeDtypeStruct((b, s, h), concat_embeddings.dtype),
        scratch_shapes=[
            pltpu.VMEM((_NBUF, _CB, s, h), concat_embeddings.dtype),
            pltpu.VMEM((_NBUF, _CB, s, h), concat_embeddings.dtype),
            pltpu.VMEM((np_, h), pos_table.dtype),
            pltpu.SemaphoreType.DMA((_NBUF,)),
            pltpu.SemaphoreType.DMA((_NBUF,)),
            pltpu.SemaphoreType.DMA,
        ],
    )(concat_embeddings, pos_table)
